# baseline (speedup 1.0000x reference)
import jax
import jax.numpy as jnp
from jax.experimental import pallas as pl
from jax.experimental.pallas import tpu as pltpu

_H = 128
_W = 128


def _quat_rot(q):
    w = q[..., 0]; x = q[..., 1]; y = q[..., 2]; z = q[..., 3]
    two_s = 2.0 / (w * w + x * x + y * y + z * z)
    xx = x * x * two_s; xy = x * y * two_s; xz = x * z * two_s
    yw = y * w * two_s; yy = y * y * two_s; yz = y * z * two_s
    zw = z * w * two_s; zz = z * z * two_s; xw = x * w * two_s
    rot = jnp.stack([1.0 - (yy + zz), xy - zw, xz + yw,
                     xy + zw, 1.0 - (xx + zz), yz - xw,
                     xz - yw, yz + xw, 1.0 - (xx + yy)], axis=-1)
    return rot.reshape(q.shape[:-1] + (3, 3))


def _raster_kernel(cand_ref, params_ref, out_ref,
                   im0_s, im1_s, im2_s, al_s, de_s):
    n = params_ref.shape[0]
    px = jax.lax.broadcasted_iota(jnp.int32, (_H, _W), 1).astype(jnp.float32)
    py = jax.lax.broadcasted_iota(jnp.int32, (_H, _W), 0).astype(jnp.float32)

    zeros = jnp.zeros((_H, _W), dtype=jnp.float32)
    im0_s[...] = zeros
    im1_s[...] = zeros
    im2_s[...] = zeros
    al_s[...] = zeros
    de_s[...] = jnp.full((_H, _W), jnp.inf, dtype=jnp.float32)

    def body(g, carry):
        @pl.when(cand_ref[g] != 0)
        def _():
            row = params_ref[pl.ds(g, 1), :]
            gu = row[:, 0:1]; gv = row[:, 1:2]
            ci00 = row[:, 2:3]; cis = row[:, 3:4]; ci11 = row[:, 4:5]
            gop = row[:, 5:6]
            c0 = row[:, 6:7]; c1 = row[:, 7:8]; c2 = row[:, 8:9]
            gz = row[:, 9:10]
            lox = row[:, 10:11]; hix = row[:, 11:12]
            loy = row[:, 12:13]; hiy = row[:, 13:14]

            depth = de_s[...]
            albuf = al_s[...]
            dx0 = px - gu
            dx1 = py - gv
            dist = ci00 * dx0 * dx0 + cis * dx0 * dx1 + ci11 * dx1 * dx1
            mask = (px >= lox) & (px < hix) & (py >= loy) & (py < hiy)
            inside = mask & (dist < 9.0) & (gz < depth)
            alpha = gop * jnp.exp(-0.5 * dist)
            na = jnp.where(inside, alpha * (1.0 - albuf), 0.0)
            one_m = 1.0 - na
            im0_s[...] = im0_s[...] * one_m + c0 * na
            im1_s[...] = im1_s[...] * one_m + c1 * na
            im2_s[...] = im2_s[...] * one_m + c2 * na
            al_s[...] = albuf + na
            de_s[...] = jnp.where(inside, gz, depth)
        return carry

    jax.lax.fori_loop(0, n, body, 0, unroll=False)
    out_ref[0, :, :] = im0_s[...]
    out_ref[1, :, :] = im1_s[...]
    out_ref[2, :, :] = im2_s[...]


def kernel(camera_pose, focal, cx, cy, image_width, image_height,
           means, scales, rotations, opacities, features):
    n = means.shape[0]
    focal_f = jnp.asarray(focal, dtype=jnp.float32)
    cx_f = jnp.asarray(cx, dtype=jnp.float32)
    cy_f = jnp.asarray(cy, dtype=jnp.float32)
    width_f = jnp.asarray(image_width, dtype=jnp.float32)
    height_f = jnp.asarray(image_height, dtype=jnp.float32)

    scales_e = jnp.exp(scales)
    rot = _quat_rot(rotations)
    opac = jax.nn.sigmoid(opacities)[:, 0]
    colors = jax.nn.sigmoid(features)
    R = camera_pose[:3, :3]
    t = camera_pose[:3, 3]
    means_cam = means @ R.T + t
    z = means_cam[:, 2]
    u = means_cam[:, 0] / z * focal_f + cx_f
    v = means_cam[:, 1] / z * focal_f + cy_f
    zero = jnp.zeros((), dtype=jnp.float32)
    one = jnp.ones((), dtype=jnp.float32)
    J = jnp.stack([jnp.stack([focal_f, zero, -cx_f]),
                   jnp.stack([zero, focal_f, -cy_f]),
                   jnp.stack([zero, zero, one])]) @ R
    V = (J[None, :, :] @ rot) * scales_e[:, None, :]
    V2 = V[:, :2, :]
    cov2d = (V2 @ jnp.swapaxes(V2, 1, 2)) / (z[:, None, None] ** 2)
    cov_inv = jnp.linalg.inv(cov2d)
    radius = jnp.max(scales_e, axis=1) * focal_f / z * 3.0

    lo_x = jnp.maximum(0.0, jnp.trunc(u - radius))
    hi_x = jnp.minimum(width_f, jnp.trunc(u + radius) + 1.0)
    lo_y = jnp.maximum(0.0, jnp.trunc(v - radius))
    hi_y = jnp.minimum(height_f, jnp.trunc(v + radius) + 1.0)

    ci00 = cov_inv[:, 0, 0]
    cis = cov_inv[:, 0, 1] + cov_inv[:, 1, 0]
    ci11 = cov_inv[:, 1, 1]

    full_bbox = (lo_x <= 0.0) & (hi_x >= _W) & (lo_y <= 0.0) & (hi_y >= _H)
    pd = (ci00 > 0.0) & (ci11 > 0.0) & (ci00 * ci11 - (0.5 * cis) ** 2 > 0.0)

    def dist_at(cpx, cpy):
        dx0 = cpx - u
        dx1 = cpy - v
        return ci00 * dx0 * dx0 + cis * dx0 * dx1 + ci11 * dx1 * dx1

    corners = ((dist_at(0.0, 0.0) < 9.0) &
               (dist_at(_W - 1.0, 0.0) < 9.0) &
               (dist_at(0.0, _H - 1.0) < 9.0) &
               (dist_at(_W - 1.0, _H - 1.0) < 9.0))
    full = full_bbox & pd & corners & jnp.isfinite(z)
    z_full = jnp.where(full, z, jnp.inf)
    pmin = jnp.concatenate([jnp.full((1,), jnp.inf, dtype=z.dtype),
                            jax.lax.cummin(z_full)[:-1]])
    nonempty = (lo_x < hi_x) & (lo_y < hi_y)
    cand = nonempty & (z < pmin)

    zf = jnp.zeros_like(u)
    cols = [u, v, ci00, cis, ci11, opac,
            colors[:, 0], colors[:, 1], colors[:, 2],
            z, lo_x, hi_x, lo_y, hi_y, zf, zf]
    params = jnp.stack(cols, axis=1)
    cand_i = cand.astype(jnp.int32)

    out = pl.pallas_call(
        _raster_kernel,
        out_shape=jax.ShapeDtypeStruct((3, _H, _W), jnp.float32),
        in_specs=[pl.BlockSpec(memory_space=pltpu.SMEM),
                  pl.BlockSpec(memory_space=pltpu.VMEM)],
        out_specs=pl.BlockSpec(memory_space=pltpu.VMEM),
        scratch_shapes=[pltpu.VMEM((_H, _W), jnp.float32)] * 5,
    )(cand_i, params)
    return jnp.transpose(out, (1, 2, 0))


# speedup vs baseline: 13.8738x; 13.8738x over previous
import jax
import jax.numpy as jnp
from jax.experimental import pallas as pl
from jax.experimental.pallas import tpu as pltpu

_H = 128
_W = 128


def _quat_rot(q):
    w = q[..., 0]; x = q[..., 1]; y = q[..., 2]; z = q[..., 3]
    two_s = 2.0 / (w * w + x * x + y * y + z * z)
    xx = x * x * two_s; xy = x * y * two_s; xz = x * z * two_s
    yw = y * w * two_s; yy = y * y * two_s; yz = y * z * two_s
    zw = z * w * two_s; zz = z * z * two_s; xw = x * w * two_s
    rot = jnp.stack([1.0 - (yy + zz), xy - zw, xz + yw,
                     xy + zw, 1.0 - (xx + zz), yz - xw,
                     xz - yw, yz + xw, 1.0 - (xx + yy)], axis=-1)
    return rot.reshape(q.shape[:-1] + (3, 3))


def _raster_kernel(cand_ref, params_ref, out_ref,
                   im0_s, im1_s, im2_s, al_s, de_s):
    n = params_ref.shape[0]
    px = jax.lax.broadcasted_iota(jnp.int32, (_H, _W), 1).astype(jnp.float32)
    py = jax.lax.broadcasted_iota(jnp.int32, (_H, _W), 0).astype(jnp.float32)

    zeros = jnp.zeros((_H, _W), dtype=jnp.float32)
    im0_s[...] = zeros
    im1_s[...] = zeros
    im2_s[...] = zeros
    al_s[...] = zeros
    de_s[...] = jnp.full((_H, _W), jnp.inf, dtype=jnp.float32)

    def body(g, carry):
        @pl.when(cand_ref[g] != 0)
        def _():
            row = params_ref[pl.ds(g, 1), :]
            gu = row[:, 0:1]; gv = row[:, 1:2]
            ci00 = row[:, 2:3]; cis = row[:, 3:4]; ci11 = row[:, 4:5]
            gop = row[:, 5:6]
            c0 = row[:, 6:7]; c1 = row[:, 7:8]; c2 = row[:, 8:9]
            gz = row[:, 9:10]
            lox = row[:, 10:11]; hix = row[:, 11:12]
            loy = row[:, 12:13]; hiy = row[:, 13:14]

            depth = de_s[...]
            albuf = al_s[...]
            dx0 = px - gu
            dx1 = py - gv
            dist = ci00 * dx0 * dx0 + cis * dx0 * dx1 + ci11 * dx1 * dx1
            mask = (px >= lox) & (px < hix) & (py >= loy) & (py < hiy)
            inside = mask & (dist < 9.0) & (gz < depth)
            alpha = gop * jnp.exp(-0.5 * dist)
            na = jnp.where(inside, alpha * (1.0 - albuf), 0.0)
            one_m = 1.0 - na
            im0_s[...] = im0_s[...] * one_m + c0 * na
            im1_s[...] = im1_s[...] * one_m + c1 * na
            im2_s[...] = im2_s[...] * one_m + c2 * na
            al_s[...] = albuf + na
            de_s[...] = jnp.where(inside, gz, depth)
        return carry

    jax.lax.fori_loop(0, n, body, 0, unroll=False)
    out_ref[0, :, :] = im0_s[...]
    out_ref[1, :, :] = im1_s[...]
    out_ref[2, :, :] = im2_s[...]


def kernel(camera_pose, focal, cx, cy, image_width, image_height,
           means, scales, rotations, opacities, features):
    n = means.shape[0]
    focal_f = jnp.asarray(focal, dtype=jnp.float32)
    cx_f = jnp.asarray(cx, dtype=jnp.float32)
    cy_f = jnp.asarray(cy, dtype=jnp.float32)
    width_f = jnp.asarray(image_width, dtype=jnp.float32)
    height_f = jnp.asarray(image_height, dtype=jnp.float32)

    scales_e = jnp.exp(scales)
    rot = _quat_rot(rotations)
    opac = jax.nn.sigmoid(opacities)[:, 0]
    colors = jax.nn.sigmoid(features)
    R = camera_pose[:3, :3]
    t = camera_pose[:3, 3]
    means_cam = means @ R.T + t
    z = means_cam[:, 2]
    u = means_cam[:, 0] / z * focal_f + cx_f
    v = means_cam[:, 1] / z * focal_f + cy_f
    zero = jnp.zeros((), dtype=jnp.float32)
    one = jnp.ones((), dtype=jnp.float32)
    J = jnp.stack([jnp.stack([focal_f, zero, -cx_f]),
                   jnp.stack([zero, focal_f, -cy_f]),
                   jnp.stack([zero, zero, one])]) @ R
    V = (J[None, :, :] @ rot) * scales_e[:, None, :]
    V2 = V[:, :2, :]
    cov2d = (V2 @ jnp.swapaxes(V2, 1, 2)) / (z[:, None, None] ** 2)
    ca = cov2d[:, 0, 0]; cb = cov2d[:, 0, 1]
    cc = cov2d[:, 1, 0]; cd = cov2d[:, 1, 1]
    det = ca * cd - cb * cc
    radius = jnp.max(scales_e, axis=1) * focal_f / z * 3.0

    lo_x = jnp.maximum(0.0, jnp.trunc(u - radius))
    hi_x = jnp.minimum(width_f, jnp.trunc(u + radius) + 1.0)
    lo_y = jnp.maximum(0.0, jnp.trunc(v - radius))
    hi_y = jnp.minimum(height_f, jnp.trunc(v + radius) + 1.0)

    ci00 = cd / det
    cis = -(cb + cc) / det
    ci11 = ca / det

    full_bbox = (lo_x <= 0.0) & (hi_x >= _W) & (lo_y <= 0.0) & (hi_y >= _H)
    pd = (ci00 > 0.0) & (ci11 > 0.0) & (ci00 * ci11 - (0.5 * cis) ** 2 > 0.0)

    def dist_at(cpx, cpy):
        dx0 = cpx - u
        dx1 = cpy - v
        return ci00 * dx0 * dx0 + cis * dx0 * dx1 + ci11 * dx1 * dx1

    corners = ((dist_at(0.0, 0.0) < 9.0) &
               (dist_at(_W - 1.0, 0.0) < 9.0) &
               (dist_at(0.0, _H - 1.0) < 9.0) &
               (dist_at(_W - 1.0, _H - 1.0) < 9.0))
    full = full_bbox & pd & corners & jnp.isfinite(z)
    z_full = jnp.where(full, z, jnp.inf)
    pmin = jnp.concatenate([jnp.full((1,), jnp.inf, dtype=z.dtype),
                            jax.lax.cummin(z_full)[:-1]])
    nonempty = (lo_x < hi_x) & (lo_y < hi_y)
    cand = nonempty & (z < pmin)

    zf = jnp.zeros_like(u)
    cols = [u, v, ci00, cis, ci11, opac,
            colors[:, 0], colors[:, 1], colors[:, 2],
            z, lo_x, hi_x, lo_y, hi_y, zf, zf]
    params = jnp.stack(cols, axis=1)
    cand_i = cand.astype(jnp.int32)

    out = pl.pallas_call(
        _raster_kernel,
        out_shape=jax.ShapeDtypeStruct((3, _H, _W), jnp.float32),
        in_specs=[pl.BlockSpec(memory_space=pltpu.SMEM),
                  pl.BlockSpec(memory_space=pltpu.VMEM)],
        out_specs=pl.BlockSpec(memory_space=pltpu.VMEM),
        scratch_shapes=[pltpu.VMEM((_H, _W), jnp.float32)] * 5,
    )(cand_i, params)
    return jnp.transpose(out, (1, 2, 0))


# speedup vs baseline: 15.2456x; 1.0989x over previous
import functools

import jax
import jax.numpy as jnp
from jax.experimental import pallas as pl
from jax.experimental.pallas import tpu as pltpu
from jax.experimental.pallas import tpu_sc as plsc

_H = 128
_W = 128


def _quat_rot(q):
    w = q[..., 0]; x = q[..., 1]; y = q[..., 2]; z = q[..., 3]
    two_s = 2.0 / (w * w + x * x + y * y + z * z)
    xx = x * x * two_s; xy = x * y * two_s; xz = x * z * two_s
    yw = y * w * two_s; yy = y * y * two_s; yz = y * z * two_s
    zw = z * w * two_s; zz = z * z * two_s; xw = x * w * two_s
    rot = jnp.stack([1.0 - (yy + zz), xy - zw, xz + yw,
                     xy + zw, 1.0 - (xx + zz), yz - xw,
                     xz - yw, yz + xw, 1.0 - (xx + yy)], axis=-1)
    return rot.reshape(q.shape[:-1] + (3, 3))


def _make_sc_raster(n, n_words):
    mesh = plsc.VectorSubcoreMesh(core_axis_name="c", subcore_axis_name="s")

    @functools.partial(
        pl.kernel, mesh=mesh,
        out_type=jax.ShapeDtypeStruct((3, _H, _W), jnp.float32),
        scratch_types=[
            pltpu.VMEM((16,), jnp.float32),
            pltpu.VMEM((n_words,), jnp.int32),
            pltpu.VMEM((3, 4, _W), jnp.float32),
            pltpu.VMEM((4, _W), jnp.float32),
            pltpu.VMEM((4, _W), jnp.float32),
        ],
    )
    def sc_raster(params_hbm, words_hbm, out_hbm,
                  row_v, words_v, img_v, al_v, de_v):
        wid = jax.lax.axis_index("s") * 2 + jax.lax.axis_index("c")
        r0 = wid * 4

        pltpu.sync_copy(words_hbm, words_v)

        fmax = jnp.float32(3.4028235e38)
        zero16 = jnp.zeros((16,), jnp.float32)
        inf16 = jnp.full((16,), fmax, jnp.float32)

        def init_j(j, c):
            r = j // 8
            col = (j % 8) * 16
            img_v[0, r, pl.ds(col, 16)] = zero16
            img_v[1, r, pl.ds(col, 16)] = zero16
            img_v[2, r, pl.ds(col, 16)] = zero16
            al_v[r, pl.ds(col, 16)] = zero16
            de_v[r, pl.ds(col, 16)] = inf16
            return c
        jax.lax.fori_loop(0, 32, init_j, 0, unroll=False)

        lane_f = jax.lax.broadcasted_iota(jnp.int32, (16,), 0).astype(
            jnp.float32)

        def draw(g):
            pltpu.sync_copy(params_hbm.at[pl.ds(g * 16, 16)], row_v)
            row = row_v[pl.ds(0, 16)]
            gu = row[0]; gv = row[1]
            ci00 = row[2]; cis = row[3]; ci11 = row[4]
            gop = row[5]
            c0 = row[6]; c1 = row[7]; c2 = row[8]
            gz = row[9]
            lox = row[10]; hix = row[11]
            loy = row[12]; hiy = row[13]

            def j_body(j, c):
                r = j // 8
                col = (j % 8) * 16
                pxv = lane_f + col.astype(jnp.float32)
                pyf = (r0 + r).astype(jnp.float32)
                dx0 = pxv - gu
                dx1 = pyf - gv
                dist = ci00 * dx0 * dx0 + cis * dx0 * dx1 + ci11 * dx1 * dx1
                depth = de_v[r, pl.ds(col, 16)]
                def ge(x, y):
                    return jnp.minimum(jnp.sign(x - y) + 1.0, 1.0)

                def lt(x, y):
                    return jnp.maximum(-jnp.sign(x - y), 0.0)

                ind = (ge(pxv, lox) * lt(pxv, hix) *
                       ge(pyf * jnp.ones((16,), jnp.float32), loy) *
                       lt(pyf * jnp.ones((16,), jnp.float32), hiy) *
                       lt(dist, 9.0) *
                       lt(gz * jnp.ones((16,), jnp.float32), depth))
                alpha = gop * jnp.exp(-0.5 * dist)
                albuf = al_v[r, pl.ds(col, 16)]
                na = alpha * (1.0 - albuf) * ind
                one_m = 1.0 - na
                img_v[0, r, pl.ds(col, 16)] = (
                    img_v[0, r, pl.ds(col, 16)] * one_m + c0 * na)
                img_v[1, r, pl.ds(col, 16)] = (
                    img_v[1, r, pl.ds(col, 16)] * one_m + c1 * na)
                img_v[2, r, pl.ds(col, 16)] = (
                    img_v[2, r, pl.ds(col, 16)] * one_m + c2 * na)
                al_v[r, pl.ds(col, 16)] = albuf + na
                de_v[r, pl.ds(col, 16)] = jnp.minimum(
                    depth, gz * ind + (1.0 - ind) * fmax)
                return c
            jax.lax.fori_loop(0, 32, j_body, 0, unroll=False)

        def grp_body(kk, c):
            wvec = words_v[pl.ds(kk * 16, 16)]
            for i in range(16):
                w = wvec[i]

                @pl.when(w != 0)
                def _(w=w, i=i):
                    def b_body(b, c2):
                        @pl.when(
                            jnp.bitwise_and(jnp.right_shift(w, b), 1) != 0)
                        def _():
                            draw((kk * 16 + i) * 32 + b)
                        return c2
                    jax.lax.fori_loop(0, 32, b_body, 0, unroll=False)
            return c
        jax.lax.fori_loop(0, n_words // 16, grp_body, 0, unroll=False)

        pltpu.sync_copy(img_v.at[0], out_hbm.at[0, pl.ds(r0, 4), :])
        pltpu.sync_copy(img_v.at[1], out_hbm.at[1, pl.ds(r0, 4), :])
        pltpu.sync_copy(img_v.at[2], out_hbm.at[2, pl.ds(r0, 4), :])

    return sc_raster


def kernel(camera_pose, focal, cx, cy, image_width, image_height,
           means, scales, rotations, opacities, features):
    n = means.shape[0]
    focal_f = jnp.asarray(focal, dtype=jnp.float32)
    cx_f = jnp.asarray(cx, dtype=jnp.float32)
    cy_f = jnp.asarray(cy, dtype=jnp.float32)
    width_f = jnp.asarray(image_width, dtype=jnp.float32)
    height_f = jnp.asarray(image_height, dtype=jnp.float32)

    scales_e = jnp.exp(scales)
    rot = _quat_rot(rotations)
    opac = jax.nn.sigmoid(opacities)[:, 0]
    colors = jax.nn.sigmoid(features)
    R = camera_pose[:3, :3]
    t = camera_pose[:3, 3]
    means_cam = means @ R.T + t
    z = means_cam[:, 2]
    u = means_cam[:, 0] / z * focal_f + cx_f
    v = means_cam[:, 1] / z * focal_f + cy_f
    zero = jnp.zeros((), dtype=jnp.float32)
    one = jnp.ones((), dtype=jnp.float32)
    J = jnp.stack([jnp.stack([focal_f, zero, -cx_f]),
                   jnp.stack([zero, focal_f, -cy_f]),
                   jnp.stack([zero, zero, one])]) @ R
    V = (J[None, :, :] @ rot) * scales_e[:, None, :]
    V2 = V[:, :2, :]
    cov2d = (V2 @ jnp.swapaxes(V2, 1, 2)) / (z[:, None, None] ** 2)
    ca = cov2d[:, 0, 0]; cb = cov2d[:, 0, 1]
    cc = cov2d[:, 1, 0]; cd = cov2d[:, 1, 1]
    det = ca * cd - cb * cc
    radius = jnp.max(scales_e, axis=1) * focal_f / z * 3.0

    lo_x = jnp.maximum(0.0, jnp.trunc(u - radius))
    hi_x = jnp.minimum(width_f, jnp.trunc(u + radius) + 1.0)
    lo_y = jnp.maximum(0.0, jnp.trunc(v - radius))
    hi_y = jnp.minimum(height_f, jnp.trunc(v + radius) + 1.0)

    ci00 = cd / det
    cis = -(cb + cc) / det
    ci11 = ca / det

    full_bbox = (lo_x <= 0.0) & (hi_x >= _W) & (lo_y <= 0.0) & (hi_y >= _H)
    pd = (ci00 > 0.0) & (ci11 > 0.0) & (ci00 * ci11 - (0.5 * cis) ** 2 > 0.0)

    def dist_at(cpx, cpy):
        dx0 = cpx - u
        dx1 = cpy - v
        return ci00 * dx0 * dx0 + cis * dx0 * dx1 + ci11 * dx1 * dx1

    corners = ((dist_at(0.0, 0.0) < 9.0) &
               (dist_at(_W - 1.0, 0.0) < 9.0) &
               (dist_at(0.0, _H - 1.0) < 9.0) &
               (dist_at(_W - 1.0, _H - 1.0) < 9.0))
    full = full_bbox & pd & corners & jnp.isfinite(z)
    z_full = jnp.where(full, z, jnp.inf)
    pmin = jnp.concatenate([jnp.full((1,), jnp.inf, dtype=z.dtype),
                            jax.lax.cummin(z_full)[:-1]])
    nonempty = (lo_x < hi_x) & (lo_y < hi_y)
    cand = nonempty & (z < pmin)

    zf = jnp.zeros_like(u)
    cols = [u, v, ci00, cis, ci11, opac,
            colors[:, 0], colors[:, 1], colors[:, 2],
            z, lo_x, hi_x, lo_y, hi_y, zf, zf]
    params = jnp.stack(cols, axis=1)

    n_words = ((n + 31) // 32 + 15) // 16 * 16
    n_pad = n_words * 32
    cand_pad = jnp.zeros((n_pad,), jnp.int32).at[:n].set(
        cand.astype(jnp.int32))
    bitvals = jnp.left_shift(jnp.int32(1), jnp.arange(32, dtype=jnp.int32))
    words = jnp.sum(cand_pad.reshape(n_words, 32) * bitvals, axis=1,
                    dtype=jnp.int32)

    out = _make_sc_raster(n, n_words)(params.reshape(-1), words)
    return jnp.transpose(out, (1, 2, 0))


# speedup vs baseline: 44.2751x; 2.9041x over previous
import functools

import jax
import jax.numpy as jnp
from jax.experimental import pallas as pl
from jax.experimental.pallas import tpu as pltpu
from jax.experimental.pallas import tpu_sc as plsc

_H = 128
_W = 128


def _quat_rot(q):
    w = q[..., 0]; x = q[..., 1]; y = q[..., 2]; z = q[..., 3]
    two_s = 2.0 / (w * w + x * x + y * y + z * z)
    xx = x * x * two_s; xy = x * y * two_s; xz = x * z * two_s
    yw = y * w * two_s; yy = y * y * two_s; yz = y * z * two_s
    zw = z * w * two_s; zz = z * z * two_s; xw = x * w * two_s
    rot = jnp.stack([1.0 - (yy + zz), xy - zw, xz + yw,
                     xy + zw, 1.0 - (xx + zz), yz - xw,
                     xz - yw, yz + xw, 1.0 - (xx + yy)], axis=-1)
    return rot.reshape(q.shape[:-1] + (3, 3))


def _make_sc_raster(n, n_words):
    mesh = plsc.VectorSubcoreMesh(core_axis_name="c", subcore_axis_name="s")

    @functools.partial(
        pl.kernel, mesh=mesh,
        out_type=jax.ShapeDtypeStruct((3, _H, _W), jnp.float32),
        scratch_types=[
            pltpu.VMEM((16,), jnp.float32),
            pltpu.VMEM((n_words,), jnp.int32),
            pltpu.VMEM((3, 4, _W), jnp.float32),
            pltpu.VMEM((4, _W), jnp.float32),
            pltpu.VMEM((4, _W), jnp.float32),
        ],
    )
    def sc_raster(params_hbm, words_hbm, out_hbm,
                  row_v, words_v, img_v, al_v, de_v):
        wid = jax.lax.axis_index("s") * 2 + jax.lax.axis_index("c")
        r0 = wid * 4

        pltpu.sync_copy(words_hbm, words_v)

        fmax = jnp.float32(3.4028235e38)
        zero16 = jnp.zeros((16,), jnp.float32)
        inf16 = jnp.full((16,), fmax, jnp.float32)

        def init_j(j, c):
            r = j // 8
            col = (j % 8) * 16
            img_v[0, r, pl.ds(col, 16)] = zero16
            img_v[1, r, pl.ds(col, 16)] = zero16
            img_v[2, r, pl.ds(col, 16)] = zero16
            al_v[r, pl.ds(col, 16)] = zero16
            de_v[r, pl.ds(col, 16)] = inf16
            return c
        jax.lax.fori_loop(0, 32, init_j, 0, unroll=False)

        lane_f = jax.lax.broadcasted_iota(jnp.int32, (16,), 0).astype(
            jnp.float32)

        def draw(g):
            pltpu.sync_copy(params_hbm.at[pl.ds(g * 16, 16)], row_v)
            row = row_v[pl.ds(0, 16)]
            gu = row[0]; gv = row[1]
            ci00 = row[2]; cis = row[3]; ci11 = row[4]
            gop = row[5]
            c0 = row[6]; c1 = row[7]; c2 = row[8]
            gz = row[9]
            lox = row[10]; hix = row[11]
            loy = row[12]; hiy = row[13]

            def j_body(j, c):
                r = j // 8
                col = (j % 8) * 16
                pxv = lane_f + col.astype(jnp.float32)
                pyf = (r0 + r).astype(jnp.float32)
                dx0 = pxv - gu
                dx1 = pyf - gv
                dist = ci00 * dx0 * dx0 + cis * dx0 * dx1 + ci11 * dx1 * dx1
                depth = de_v[r, pl.ds(col, 16)]
                def ge(x, y):
                    return jnp.minimum(jnp.sign(x - y) + 1.0, 1.0)

                def lt(x, y):
                    return jnp.maximum(-jnp.sign(x - y), 0.0)

                ind = (ge(pxv, lox) * lt(pxv, hix) *
                       ge(pyf * jnp.ones((16,), jnp.float32), loy) *
                       lt(pyf * jnp.ones((16,), jnp.float32), hiy) *
                       lt(dist, 9.0) *
                       lt(gz * jnp.ones((16,), jnp.float32), depth))
                alpha = gop * jnp.exp(-0.5 * dist)
                albuf = al_v[r, pl.ds(col, 16)]
                na = alpha * (1.0 - albuf) * ind
                one_m = 1.0 - na
                img_v[0, r, pl.ds(col, 16)] = (
                    img_v[0, r, pl.ds(col, 16)] * one_m + c0 * na)
                img_v[1, r, pl.ds(col, 16)] = (
                    img_v[1, r, pl.ds(col, 16)] * one_m + c1 * na)
                img_v[2, r, pl.ds(col, 16)] = (
                    img_v[2, r, pl.ds(col, 16)] * one_m + c2 * na)
                al_v[r, pl.ds(col, 16)] = albuf + na
                de_v[r, pl.ds(col, 16)] = jnp.minimum(
                    depth, gz * ind + (1.0 - ind) * fmax)
                return c
            jax.lax.fori_loop(0, 32, j_body, 0, unroll=False)

        def grp_body(kk, c):
            wvec = words_v[pl.ds(kk * 16, 16)]
            for i in range(16):
                w = wvec[i]

                @pl.when(w != 0)
                def _(w=w, i=i):
                    def b_body(b, c2):
                        @pl.when(
                            jnp.bitwise_and(jnp.right_shift(w, b), 1) != 0)
                        def _():
                            draw((kk * 16 + i) * 32 + b)
                        return c2
                    jax.lax.fori_loop(0, 32, b_body, 0, unroll=False)
            return c
        jax.lax.fori_loop(0, n_words // 16, grp_body, 0, unroll=False)

        pltpu.sync_copy(img_v.at[0], out_hbm.at[0, pl.ds(r0, 4), :])
        pltpu.sync_copy(img_v.at[1], out_hbm.at[1, pl.ds(r0, 4), :])
        pltpu.sync_copy(img_v.at[2], out_hbm.at[2, pl.ds(r0, 4), :])

    return sc_raster


def kernel(camera_pose, focal, cx, cy, image_width, image_height,
           means, scales, rotations, opacities, features):
    n = means.shape[0]
    focal_f = jnp.asarray(focal, dtype=jnp.float32)
    cx_f = jnp.asarray(cx, dtype=jnp.float32)
    cy_f = jnp.asarray(cy, dtype=jnp.float32)
    width_f = jnp.asarray(image_width, dtype=jnp.float32)
    height_f = jnp.asarray(image_height, dtype=jnp.float32)

    scales_e = jnp.exp(scales)
    s0 = scales_e[:, 0]; s1 = scales_e[:, 1]; s2 = scales_e[:, 2]
    opac = jax.nn.sigmoid(opacities)[:, 0]
    colors = jax.nn.sigmoid(features)

    qw = rotations[:, 0]; qx = rotations[:, 1]
    qy = rotations[:, 2]; qz = rotations[:, 3]
    two_s = 2.0 / (qw * qw + qx * qx + qy * qy + qz * qz)
    xx = qx * qx * two_s; xy = qx * qy * two_s; xz = qx * qz * two_s
    yw = qy * qw * two_s; yy = qy * qy * two_s; yz = qy * qz * two_s
    zw = qz * qw * two_s; zz = qz * qz * two_s; xw = qx * qw * two_s
    m00 = 1.0 - (yy + zz); m01 = xy - zw; m02 = xz + yw
    m10 = xy + zw; m11 = 1.0 - (xx + zz); m12 = yz - xw
    m20 = xz - yw; m21 = yz + xw; m22 = 1.0 - (xx + yy)

    r00 = camera_pose[0, 0]; r01 = camera_pose[0, 1]; r02 = camera_pose[0, 2]
    r10 = camera_pose[1, 0]; r11 = camera_pose[1, 1]; r12 = camera_pose[1, 2]
    r20 = camera_pose[2, 0]; r21 = camera_pose[2, 1]; r22 = camera_pose[2, 2]
    t0 = camera_pose[0, 3]; t1 = camera_pose[1, 3]; t2 = camera_pose[2, 3]

    p0 = means[:, 0]; p1 = means[:, 1]; p2 = means[:, 2]
    xc = p0 * r00 + p1 * r01 + p2 * r02 + t0
    yc = p0 * r10 + p1 * r11 + p2 * r12 + t1
    z = p0 * r20 + p1 * r21 + p2 * r22 + t2
    u = xc / z * focal_f + cx_f
    v = yc / z * focal_f + cy_f

    j00 = focal_f * r00 - cx_f * r20
    j01 = focal_f * r01 - cx_f * r21
    j02 = focal_f * r02 - cx_f * r22
    j10 = focal_f * r10 - cy_f * r20
    j11 = focal_f * r11 - cy_f * r21
    j12 = focal_f * r12 - cy_f * r22

    v00 = (j00 * m00 + j01 * m10 + j02 * m20) * s0
    v01 = (j00 * m01 + j01 * m11 + j02 * m21) * s1
    v02 = (j00 * m02 + j01 * m12 + j02 * m22) * s2
    v10 = (j10 * m00 + j11 * m10 + j12 * m20) * s0
    v11 = (j10 * m01 + j11 * m11 + j12 * m21) * s1
    v12 = (j10 * m02 + j11 * m12 + j12 * m22) * s2

    z2 = z * z
    ca = (v00 * v00 + v01 * v01 + v02 * v02) / z2
    cb = (v00 * v10 + v01 * v11 + v02 * v12) / z2
    cd = (v10 * v10 + v11 * v11 + v12 * v12) / z2
    cc = cb
    det = ca * cd - cb * cc
    radius = jnp.maximum(jnp.maximum(s0, s1), s2) * focal_f / z * 3.0

    lo_x = jnp.maximum(0.0, jnp.trunc(u - radius))
    hi_x = jnp.minimum(width_f, jnp.trunc(u + radius) + 1.0)
    lo_y = jnp.maximum(0.0, jnp.trunc(v - radius))
    hi_y = jnp.minimum(height_f, jnp.trunc(v + radius) + 1.0)

    ci00 = cd / det
    cis = -(cb + cc) / det
    ci11 = ca / det

    full_bbox = (lo_x <= 0.0) & (hi_x >= _W) & (lo_y <= 0.0) & (hi_y >= _H)
    pd = (ci00 > 0.0) & (ci11 > 0.0) & (ci00 * ci11 - (0.5 * cis) ** 2 > 0.0)

    def dist_at(cpx, cpy):
        dx0 = cpx - u
        dx1 = cpy - v
        return ci00 * dx0 * dx0 + cis * dx0 * dx1 + ci11 * dx1 * dx1

    corners = ((dist_at(0.0, 0.0) < 9.0) &
               (dist_at(_W - 1.0, 0.0) < 9.0) &
               (dist_at(0.0, _H - 1.0) < 9.0) &
               (dist_at(_W - 1.0, _H - 1.0) < 9.0))
    full = full_bbox & pd & corners & jnp.isfinite(z)
    z_full = jnp.where(full, z, jnp.inf)
    pmin = jnp.concatenate([jnp.full((1,), jnp.inf, dtype=z.dtype),
                            jax.lax.cummin(z_full)[:-1]])
    nonempty = (lo_x < hi_x) & (lo_y < hi_y)
    cand = nonempty & (z < pmin)

    zf = jnp.zeros_like(u)
    cols = [u, v, ci00, cis, ci11, opac,
            colors[:, 0], colors[:, 1], colors[:, 2],
            z, lo_x, hi_x, lo_y, hi_y, zf, zf]
    params = jnp.stack(cols, axis=1)

    n_words = ((n + 31) // 32 + 15) // 16 * 16
    n_pad = n_words * 32
    cand_pad = jnp.zeros((n_pad,), jnp.int32).at[:n].set(
        cand.astype(jnp.int32))
    bitvals = jnp.left_shift(jnp.int32(1), jnp.arange(32, dtype=jnp.int32))
    words = jnp.sum(cand_pad.reshape(n_words, 32) * bitvals, axis=1,
                    dtype=jnp.int32)

    out = _make_sc_raster(n, n_words)(params.reshape(-1), words)
    return jnp.transpose(out, (1, 2, 0))


# speedup vs baseline: 52.9445x; 1.1958x over previous
import functools

import jax
import jax.numpy as jnp
from jax.experimental import pallas as pl
from jax.experimental.pallas import tpu as pltpu
from jax.experimental.pallas import tpu_sc as plsc

_H = 128
_W = 128


def _quat_rot(q):
    w = q[..., 0]; x = q[..., 1]; y = q[..., 2]; z = q[..., 3]
    two_s = 2.0 / (w * w + x * x + y * y + z * z)
    xx = x * x * two_s; xy = x * y * two_s; xz = x * z * two_s
    yw = y * w * two_s; yy = y * y * two_s; yz = y * z * two_s
    zw = z * w * two_s; zz = z * z * two_s; xw = x * w * two_s
    rot = jnp.stack([1.0 - (yy + zz), xy - zw, xz + yw,
                     xy + zw, 1.0 - (xx + zz), yz - xw,
                     xz - yw, yz + xw, 1.0 - (xx + yy)], axis=-1)
    return rot.reshape(q.shape[:-1] + (3, 3))


def _make_sc_raster(n, n_words):
    mesh = plsc.VectorSubcoreMesh(core_axis_name="c", subcore_axis_name="s")

    @functools.partial(
        pl.kernel, mesh=mesh,
        out_type=jax.ShapeDtypeStruct((3, _H, _W), jnp.float32),
        scratch_types=[
            pltpu.VMEM((16,), jnp.float32),
            pltpu.VMEM((n_words,), jnp.int32),
            pltpu.SMEM((n_words,), jnp.int32),
            pltpu.VMEM((3, 4, _W), jnp.float32),
            pltpu.VMEM((4, _W), jnp.float32),
            pltpu.VMEM((4, _W), jnp.float32),
        ],
    )
    def sc_raster(params_hbm, words_hbm, out_hbm,
                  row_v, words_v, words_s, img_v, al_v, de_v):
        wid = jax.lax.axis_index("s") * 2 + jax.lax.axis_index("c")
        r0 = wid * 4

        pltpu.sync_copy(words_hbm, words_v)

        fmax = jnp.float32(3.4028235e38)
        zero16 = jnp.zeros((16,), jnp.float32)
        inf16 = jnp.full((16,), fmax, jnp.float32)

        def init_j(j, c):
            r = j // 8
            col = (j % 8) * 16
            img_v[0, r, pl.ds(col, 16)] = zero16
            img_v[1, r, pl.ds(col, 16)] = zero16
            img_v[2, r, pl.ds(col, 16)] = zero16
            al_v[r, pl.ds(col, 16)] = zero16
            de_v[r, pl.ds(col, 16)] = inf16
            return c
        jax.lax.fori_loop(0, 32, init_j, 0, unroll=False)

        lane_f = jax.lax.broadcasted_iota(jnp.int32, (16,), 0).astype(
            jnp.float32)
        pxs = [lane_f + jnp.float32(16.0 * jc) for jc in range(8)]
        r0f = r0.astype(jnp.float32)

        def ge(x, y):
            return jnp.minimum(jnp.sign(x - y) + 1.0, 1.0)

        def lt(x, y):
            return jnp.maximum(-jnp.sign(x - y), 0.0)

        def draw(g):
            pltpu.sync_copy(params_hbm.at[pl.ds(g * 16, 16)], row_v)
            row = row_v[pl.ds(0, 16)]
            gu = row[0]; gv = row[1]
            ci00 = row[2]; cis = row[3]; ci11 = row[4]
            gop = row[5]
            c0 = row[6]; c1 = row[7]; c2 = row[8]
            gz = row[9]
            lox = row[10]; hix = row[11]
            loy = row[12]; hiy = row[13]

            for r in range(4):
                pyf = r0f + jnp.float32(r)
                pyv = zero16 + pyf
                indy = ge(pyv, loy) * lt(pyv, hiy)
                if True:
                    dx1 = pyf - gv
                    b_r = cis * dx1
                    c_r = (ci11 * dx1) * dx1
                    for jc in range(8):
                        col = jc * 16
                        pxv = pxs[jc]
                        dx0 = pxv - gu
                        dist = (ci00 * dx0 + b_r) * dx0 + c_r
                        depth = de_v[r, pl.ds(col, 16)]
                        ind = (indy * ge(pxv, lox) * lt(pxv, hix) *
                               lt(dist, 9.0) * lt(gz - depth, 0.0))
                        alpha = gop * jnp.exp(-0.5 * dist)
                        albuf = al_v[r, pl.ds(col, 16)]
                        na = (alpha * (1.0 - albuf)) * ind
                        one_m = 1.0 - na
                        img_v[0, r, pl.ds(col, 16)] = (
                            img_v[0, r, pl.ds(col, 16)] * one_m + c0 * na)
                        img_v[1, r, pl.ds(col, 16)] = (
                            img_v[1, r, pl.ds(col, 16)] * one_m + c1 * na)
                        img_v[2, r, pl.ds(col, 16)] = (
                            img_v[2, r, pl.ds(col, 16)] * one_m + c2 * na)
                        al_v[r, pl.ds(col, 16)] = albuf + na
                        de_v[r, pl.ds(col, 16)] = jnp.minimum(
                            depth, gz * ind + (1.0 - ind) * fmax)

        def stage_body(kk, c):
            wvec = words_v[pl.ds(kk * 16, 16)]
            for i in range(16):
                words_s[kk * 16 + i] = wvec[i]
            return c
        jax.lax.fori_loop(0, n_words // 16, stage_body, 0, unroll=False)

        def w_body(k, c):
            w = words_s[k]

            @pl.when(w != 0)
            def _():
                def b_body(b, c2):
                    @pl.when(
                        jnp.bitwise_and(jnp.right_shift(w, b), 1) != 0)
                    def _():
                        draw(k * 32 + b)
                    return c2
                jax.lax.fori_loop(0, 32, b_body, 0, unroll=False)
            return c
        jax.lax.fori_loop(0, n_words, w_body, 0, unroll=False)

        pltpu.sync_copy(img_v.at[0], out_hbm.at[0, pl.ds(r0, 4), :])
        pltpu.sync_copy(img_v.at[1], out_hbm.at[1, pl.ds(r0, 4), :])
        pltpu.sync_copy(img_v.at[2], out_hbm.at[2, pl.ds(r0, 4), :])

    return sc_raster


def kernel(camera_pose, focal, cx, cy, image_width, image_height,
           means, scales, rotations, opacities, features):
    n = means.shape[0]
    focal_f = jnp.asarray(focal, dtype=jnp.float32)
    cx_f = jnp.asarray(cx, dtype=jnp.float32)
    cy_f = jnp.asarray(cy, dtype=jnp.float32)
    width_f = jnp.asarray(image_width, dtype=jnp.float32)
    height_f = jnp.asarray(image_height, dtype=jnp.float32)

    scales_e = jnp.exp(scales)
    s0 = scales_e[:, 0]; s1 = scales_e[:, 1]; s2 = scales_e[:, 2]
    opac = jax.nn.sigmoid(opacities)[:, 0]
    colors = jax.nn.sigmoid(features)

    qw = rotations[:, 0]; qx = rotations[:, 1]
    qy = rotations[:, 2]; qz = rotations[:, 3]
    two_s = 2.0 / (qw * qw + qx * qx + qy * qy + qz * qz)
    xx = qx * qx * two_s; xy = qx * qy * two_s; xz = qx * qz * two_s
    yw = qy * qw * two_s; yy = qy * qy * two_s; yz = qy * qz * two_s
    zw = qz * qw * two_s; zz = qz * qz * two_s; xw = qx * qw * two_s
    m00 = 1.0 - (yy + zz); m01 = xy - zw; m02 = xz + yw
    m10 = xy + zw; m11 = 1.0 - (xx + zz); m12 = yz - xw
    m20 = xz - yw; m21 = yz + xw; m22 = 1.0 - (xx + yy)

    r00 = camera_pose[0, 0]; r01 = camera_pose[0, 1]; r02 = camera_pose[0, 2]
    r10 = camera_pose[1, 0]; r11 = camera_pose[1, 1]; r12 = camera_pose[1, 2]
    r20 = camera_pose[2, 0]; r21 = camera_pose[2, 1]; r22 = camera_pose[2, 2]
    t0 = camera_pose[0, 3]; t1 = camera_pose[1, 3]; t2 = camera_pose[2, 3]

    p0 = means[:, 0]; p1 = means[:, 1]; p2 = means[:, 2]
    xc = p0 * r00 + p1 * r01 + p2 * r02 + t0
    yc = p0 * r10 + p1 * r11 + p2 * r12 + t1
    z = p0 * r20 + p1 * r21 + p2 * r22 + t2
    u = xc / z * focal_f + cx_f
    v = yc / z * focal_f + cy_f

    j00 = focal_f * r00 - cx_f * r20
    j01 = focal_f * r01 - cx_f * r21
    j02 = focal_f * r02 - cx_f * r22
    j10 = focal_f * r10 - cy_f * r20
    j11 = focal_f * r11 - cy_f * r21
    j12 = focal_f * r12 - cy_f * r22

    v00 = (j00 * m00 + j01 * m10 + j02 * m20) * s0
    v01 = (j00 * m01 + j01 * m11 + j02 * m21) * s1
    v02 = (j00 * m02 + j01 * m12 + j02 * m22) * s2
    v10 = (j10 * m00 + j11 * m10 + j12 * m20) * s0
    v11 = (j10 * m01 + j11 * m11 + j12 * m21) * s1
    v12 = (j10 * m02 + j11 * m12 + j12 * m22) * s2

    z2 = z * z
    ca = (v00 * v00 + v01 * v01 + v02 * v02) / z2
    cb = (v00 * v10 + v01 * v11 + v02 * v12) / z2
    cd = (v10 * v10 + v11 * v11 + v12 * v12) / z2
    cc = cb
    det = ca * cd - cb * cc
    radius = jnp.maximum(jnp.maximum(s0, s1), s2) * focal_f / z * 3.0

    lo_x = jnp.maximum(0.0, jnp.trunc(u - radius))
    hi_x = jnp.minimum(width_f, jnp.trunc(u + radius) + 1.0)
    lo_y = jnp.maximum(0.0, jnp.trunc(v - radius))
    hi_y = jnp.minimum(height_f, jnp.trunc(v + radius) + 1.0)

    ci00 = cd / det
    cis = -(cb + cc) / det
    ci11 = ca / det

    full_bbox = (lo_x <= 0.0) & (hi_x >= _W) & (lo_y <= 0.0) & (hi_y >= _H)
    pd = (ci00 > 0.0) & (ci11 > 0.0) & (ci00 * ci11 - (0.5 * cis) ** 2 > 0.0)

    def dist_at(cpx, cpy):
        dx0 = cpx - u
        dx1 = cpy - v
        return ci00 * dx0 * dx0 + cis * dx0 * dx1 + ci11 * dx1 * dx1

    corners = ((dist_at(0.0, 0.0) < 9.0) &
               (dist_at(_W - 1.0, 0.0) < 9.0) &
               (dist_at(0.0, _H - 1.0) < 9.0) &
               (dist_at(_W - 1.0, _H - 1.0) < 9.0))
    full = full_bbox & pd & corners & jnp.isfinite(z)
    z_full = jnp.where(full, z, jnp.inf)
    pmin = jnp.concatenate([jnp.full((1,), jnp.inf, dtype=z.dtype),
                            jax.lax.cummin(z_full)[:-1]])
    nonempty = (lo_x < hi_x) & (lo_y < hi_y)
    cand = nonempty & (z < pmin)

    zf = jnp.zeros_like(u)
    cols = [u, v, ci00, cis, ci11, opac,
            colors[:, 0], colors[:, 1], colors[:, 2],
            z, lo_x, hi_x, lo_y, hi_y, zf, zf]
    params = jnp.stack(cols, axis=1)

    n_words = ((n + 31) // 32 + 15) // 16 * 16
    n_pad = n_words * 32
    cand_pad = jnp.zeros((n_pad,), jnp.int32).at[:n].set(
        cand.astype(jnp.int32))
    bitvals = jnp.left_shift(jnp.int32(1), jnp.arange(32, dtype=jnp.int32))
    words = jnp.sum(cand_pad.reshape(n_words, 32) * bitvals, axis=1,
                    dtype=jnp.int32)

    out = _make_sc_raster(n, n_words)(params.reshape(-1), words)
    return jnp.transpose(out, (1, 2, 0))
